# baseline (speedup 1.0000x reference)
import functools

import jax
import jax.numpy as jnp
from jax import lax
from jax.experimental import pallas as pl
from jax.experimental.pallas import tpu as pltpu
from jax.experimental.pallas import tpu_sc as plsc

NC = 2
NS = 16
NW = NC * NS
L = 16

B = 16384
D = 32
BPW = B // NW
CHUNK = 128
NCH = BPW // CHUNK

def _mf_body(user_r, item_r, ue_r, ie_r, ub_r, ib_r, out_r,
             idx_u, idx_i, rows_u, rows_i, bu, bi, preds, sem):
  wid = lax.axis_index("s") * NC + lax.axis_index("c")
  base = wid * BPW

  for c in range(NCH):
    pltpu.sync_copy(user_r.at[pl.ds(base + c * CHUNK, CHUNK)], idx_u.at[c])
    pltpu.sync_copy(item_r.at[pl.ds(base + c * CHUNK, CHUNK)], idx_i.at[c])

  copies = []
  for c in range(NCH):
    copies.append(pltpu.async_copy(ue_r.at[idx_u.at[c]], rows_u.at[c], sem))
    copies.append(pltpu.async_copy(ie_r.at[idx_i.at[c]], rows_i.at[c], sem))
    copies.append(pltpu.async_copy(ub_r.at[idx_u.at[c]], bu.at[c], sem))
    copies.append(pltpu.async_copy(ib_r.at[idx_i.at[c]], bi.at[c], sem))
  for cp in copies:
    cp.wait()

  lane = lax.iota(jnp.int32, L)
  zero = jnp.zeros((L,), jnp.int32)
  for c in range(NCH):
    cvec = jnp.full((L,), c, jnp.int32)
    def group_body(g, _, c=c, cvec=cvec):
      acc = jnp.zeros((L,), jnp.float32)
      for u in range(L):
        r = g * L + u
        p = (rows_u[c, r, pl.ds(0, L)] * rows_i[c, r, pl.ds(0, L)]
             + rows_u[c, r, pl.ds(L, L)] * rows_i[c, r, pl.ds(L, L)])
        acc = jnp.where(lane == u, jnp.sum(p), acc)
      rvec = g * L + lane
      bu_v = plsc.load_gather(bu, [cvec, rvec, zero])
      bi_v = plsc.load_gather(bi, [cvec, rvec, zero])
      x = acc + bu_v + bi_v
      preds[pl.ds(c * CHUNK + g * L, L)] = 1.0 / (1.0 + jnp.exp(-x))
      return 0
    lax.fori_loop(0, CHUNK // L, group_body, 0)

  pltpu.sync_copy(preds, out_r.at[pl.ds(base, BPW)])


@jax.jit
def kernel(user, item, user_e, item_e, user_b, item_b):
  user = user.astype(jnp.int32)
  item = item.astype(jnp.int32)

  mesh = plsc.VectorSubcoreMesh(core_axis_name="c", subcore_axis_name="s")
  params = pltpu.CompilerParams(
      needs_layout_passes=False, use_tc_tiling_on_sc=False)

  ones32 = jnp.ones((1, D), jnp.float32)
  ub16 = user_b * ones32
  ib16 = item_b * ones32

  run = pl.kernel(
      _mf_body,
      out_type=jax.ShapeDtypeStruct((B,), jnp.float32),
      mesh=mesh,
      compiler_params=params,
      scratch_types=[
          pltpu.VMEM((NCH, CHUNK), jnp.int32),
          pltpu.VMEM((NCH, CHUNK), jnp.int32),
          pltpu.VMEM((NCH, CHUNK, D), jnp.float32),
          pltpu.VMEM((NCH, CHUNK, D), jnp.float32),
          pltpu.VMEM((NCH, CHUNK, D), jnp.float32),
          pltpu.VMEM((NCH, CHUNK, D), jnp.float32),
          pltpu.VMEM((BPW,), jnp.float32),
          pltpu.SemaphoreType.DMA,
      ],
  )
  return run(user, item, user_e, item_e, ub16, ib16)


# speedup vs baseline: 2.4778x; 2.4778x over previous
import jax
import jax.numpy as jnp
from jax import lax
from jax.experimental import pallas as pl
from jax.experimental.pallas import tpu as pltpu
from jax.experimental.pallas import tpu_sc as plsc

NC = 2
NS = 16
NW = NC * NS
L = 16

B = 16384
D = 32
BPW = B // NW
CHUNK = 128
NCH = BPW // CHUNK

NROWS = 1000000
TBL_R = NROWS * D // 128
BIAS_R = 8192


def _mf_body(user_r, item_r, ue_r, ie_r, ub_r, ib_r, out_r,
             idx_u, idx_i, idxe_u, idxe_i, idxb_u, idxb_i,
             ru, ri, rbu, rbi, preds, sem):
  wid = lax.axis_index("s") * NC + lax.axis_index("c")
  base = wid * BPW

  for c in range(NCH):
    pltpu.sync_copy(user_r.at[pl.ds(base + c * CHUNK, CHUNK)], idx_u.at[c])
    pltpu.sync_copy(item_r.at[pl.ds(base + c * CHUNK, CHUNK)], idx_i.at[c])

  for c in range(NCH):
    def hi_body(g, _, c=c):
      sl = pl.ds(g * L, L)
      idxe_u[c, sl] = lax.shift_right_logical(idx_u[c, sl], 2)
      idxe_i[c, sl] = lax.shift_right_logical(idx_i[c, sl], 2)
      idxb_u[c, sl] = lax.shift_right_logical(idx_u[c, sl], 7)
      idxb_i[c, sl] = lax.shift_right_logical(idx_i[c, sl], 7)
      return 0
    lax.fori_loop(0, CHUNK // L, hi_body, 0)

  lane = lax.iota(jnp.int32, L)
  m3 = jnp.full((L,), 3, jnp.int32)
  m127 = jnp.full((L,), 127, jnp.int32)

  for c in range(NCH):
    cps = [
        pltpu.async_copy(ue_r.at[idxe_u.at[c]], ru, sem),
        pltpu.async_copy(ie_r.at[idxe_i.at[c]], ri, sem),
        pltpu.async_copy(ub_r.at[idxb_u.at[c]], rbu, sem),
        pltpu.async_copy(ib_r.at[idxb_i.at[c]], rbi, sem),
    ]
    for cp in cps:
      cp.wait()

    def group_body(g, _, c=c):
      sl = pl.ds(g * L, L)
      low_u = (idx_u[c, sl] & m3) * D
      low_i = (idx_i[c, sl] & m3) * D
      acc = jnp.zeros((L,), jnp.float32)
      for u in range(L):
        r = g * L + u
        rvec = jnp.full((L,), 0, jnp.int32) + r
        cu = low_u[u] + lane
        ci = low_i[u] + lane
        p = (plsc.load_gather(ru, [rvec, cu])
             * plsc.load_gather(ri, [rvec, ci])
             + plsc.load_gather(ru, [rvec, cu + L])
             * plsc.load_gather(ri, [rvec, ci + L]))
        acc = jnp.where(lane == u, jnp.sum(p), acc)
      rows = g * L + lane
      bu_v = plsc.load_gather(rbu, [rows, idx_u[c, sl] & m127])
      bi_v = plsc.load_gather(rbi, [rows, idx_i[c, sl] & m127])
      x = acc + bu_v + bi_v
      preds[pl.ds(c * CHUNK + g * L, L)] = 1.0 / (1.0 + jnp.exp(-x))
      return 0
    lax.fori_loop(0, CHUNK // L, group_body, 0)

  pltpu.sync_copy(preds, out_r.at[pl.ds(base, BPW)])


@jax.jit
def kernel(user, item, user_e, item_e, user_b, item_b):
  user = user.astype(jnp.int32)
  item = item.astype(jnp.int32)

  ue2 = user_e.reshape(TBL_R, 128)
  ie2 = item_e.reshape(TBL_R, 128)
  ubp = jnp.pad(user_b[:, 0], (0, BIAS_R * 128 - NROWS)).reshape(BIAS_R, 128)
  ibp = jnp.pad(item_b[:, 0], (0, BIAS_R * 128 - NROWS)).reshape(BIAS_R, 128)

  mesh = plsc.VectorSubcoreMesh(core_axis_name="c", subcore_axis_name="s")
  run = pl.kernel(
      _mf_body,
      out_type=jax.ShapeDtypeStruct((B,), jnp.float32),
      mesh=mesh,
      compiler_params=pltpu.CompilerParams(
          needs_layout_passes=False, use_tc_tiling_on_sc=True),
      scratch_types=[
          pltpu.VMEM((NCH, CHUNK), jnp.int32),
          pltpu.VMEM((NCH, CHUNK), jnp.int32),
          pltpu.VMEM((NCH, CHUNK), jnp.int32),
          pltpu.VMEM((NCH, CHUNK), jnp.int32),
          pltpu.VMEM((NCH, CHUNK), jnp.int32),
          pltpu.VMEM((NCH, CHUNK), jnp.int32),
          pltpu.VMEM((CHUNK, 128), jnp.float32),
          pltpu.VMEM((CHUNK, 128), jnp.float32),
          pltpu.VMEM((CHUNK, 128), jnp.float32),
          pltpu.VMEM((CHUNK, 128), jnp.float32),
          pltpu.VMEM((BPW,), jnp.float32),
          pltpu.SemaphoreType.DMA,
      ],
  )
  return run(user, item, ue2, ie2, ubp, ibp)


# speedup vs baseline: 2.4855x; 1.0031x over previous
import jax
import jax.numpy as jnp
from jax import lax
from jax.experimental import pallas as pl
from jax.experimental.pallas import tpu as pltpu
from jax.experimental.pallas import tpu_sc as plsc

NC = 2
NS = 16
NW = NC * NS
L = 16

B = 16384
D = 32
BPW = B // NW
CHUNK = 128
NCH = BPW // CHUNK

NROWS = 1000000
TBL_R = NROWS * D // 128
BIAS_R = 8192


def _mf_body(user_r, item_r, ue_r, ie_r, ub_r, ib_r, out_r,
             idx_u, idx_i, idxb_u, idxb_i,
             ru, ri, rbu, rbi, preds, sem):
  wid = lax.axis_index("s") * NC + lax.axis_index("c")
  base = wid * BPW

  for c in range(NCH):
    pltpu.sync_copy(user_r.at[pl.ds(base + c * CHUNK, CHUNK)], idx_u.at[c])
    pltpu.sync_copy(item_r.at[pl.ds(base + c * CHUNK, CHUNK)], idx_i.at[c])

  for c in range(NCH):
    def hi_body(g, _, c=c):
      sl = pl.ds(g * L, L)
      idxb_u[c, sl] = lax.shift_right_logical(idx_u[c, sl], 7)
      idxb_i[c, sl] = lax.shift_right_logical(idx_i[c, sl], 7)
      return 0
    lax.fori_loop(0, CHUNK // L, hi_body, 0)

  lane = lax.iota(jnp.int32, L)
  m127 = jnp.full((L,), 127, jnp.int32)

  for c in range(NCH):
    cps = [
        pltpu.async_copy(ue_r.at[idx_u.at[c]], ru, sem),
        pltpu.async_copy(ie_r.at[idx_i.at[c]], ri, sem),
        pltpu.async_copy(ub_r.at[idxb_u.at[c]], rbu, sem),
        pltpu.async_copy(ib_r.at[idxb_i.at[c]], rbi, sem),
    ]
    for cp in cps:
      cp.wait()

    def group_body(g, _, c=c):
      sl = pl.ds(g * L, L)
      acc = jnp.zeros((L,), jnp.float32)
      for u in range(L):
        r = g * L + u
        p = (ru[r, pl.ds(0, L)] * ri[r, pl.ds(0, L)]
             + ru[r, pl.ds(L, L)] * ri[r, pl.ds(L, L)])
        acc = jnp.where(lane == u, jnp.sum(p), acc)
      rows = g * L + lane
      bu_v = plsc.load_gather(rbu, [rows, idx_u[c, sl] & m127])
      bi_v = plsc.load_gather(rbi, [rows, idx_i[c, sl] & m127])
      x = acc + bu_v + bi_v
      preds[pl.ds(c * CHUNK + g * L, L)] = 1.0 / (1.0 + jnp.exp(-x))
      return 0
    lax.fori_loop(0, CHUNK // L, group_body, 0)

  pltpu.sync_copy(preds, out_r.at[pl.ds(base, BPW)])


@jax.jit
def kernel(user, item, user_e, item_e, user_b, item_b):
  user = user.astype(jnp.int32)
  item = item.astype(jnp.int32)

  ubp = jnp.pad(user_b[:, 0], (0, BIAS_R * 128 - NROWS)).reshape(BIAS_R, 128)
  ibp = jnp.pad(item_b[:, 0], (0, BIAS_R * 128 - NROWS)).reshape(BIAS_R, 128)

  mesh = plsc.VectorSubcoreMesh(core_axis_name="c", subcore_axis_name="s")
  run = pl.kernel(
      _mf_body,
      out_type=jax.ShapeDtypeStruct((B,), jnp.float32),
      mesh=mesh,
      compiler_params=pltpu.CompilerParams(
          needs_layout_passes=False, use_tc_tiling_on_sc=False),
      scratch_types=[
          pltpu.VMEM((NCH, CHUNK), jnp.int32),
          pltpu.VMEM((NCH, CHUNK), jnp.int32),
          pltpu.VMEM((NCH, CHUNK), jnp.int32),
          pltpu.VMEM((NCH, CHUNK), jnp.int32),
          pltpu.VMEM((CHUNK, D), jnp.float32),
          pltpu.VMEM((CHUNK, D), jnp.float32),
          pltpu.VMEM((CHUNK, 128), jnp.float32),
          pltpu.VMEM((CHUNK, 128), jnp.float32),
          pltpu.VMEM((BPW,), jnp.float32),
          pltpu.SemaphoreType.DMA,
      ],
  )
  return run(user, item, user_e, item_e, ubp, ibp)


# speedup vs baseline: 2.4934x; 1.0032x over previous
import jax
import jax.numpy as jnp
from jax import lax
from jax.experimental import pallas as pl
from jax.experimental.pallas import tpu as pltpu
from jax.experimental.pallas import tpu_sc as plsc

NC = 2
NS = 16
NW = NC * NS
L = 16

B = 16384
D = 32
BPW = B // NW
CHUNK = 128
NCH = BPW // CHUNK

NROWS = 1000000
TBL_R = NROWS * D // 128
BIAS_R = 8192


def _mf_body(user_r, item_r, ue_r, ie_r, ub_r, ib_r, out_r,
             idx_u, idx_i, idxb_u, idxb_i,
             ru, ri, rbu, rbi, preds, sem):
  wid = lax.axis_index("s") * NC + lax.axis_index("c")
  base = wid * BPW

  for c in range(NCH):
    pltpu.sync_copy(user_r.at[pl.ds(base + c * CHUNK, CHUNK)], idx_u.at[c])
    pltpu.sync_copy(item_r.at[pl.ds(base + c * CHUNK, CHUNK)], idx_i.at[c])

  for c in range(NCH):
    def hi_body(g, _, c=c):
      sl = pl.ds(g * L, L)
      idxb_u[c, sl] = lax.shift_right_logical(idx_u[c, sl], 7)
      idxb_i[c, sl] = lax.shift_right_logical(idx_i[c, sl], 7)
      return 0
    lax.fori_loop(0, CHUNK // L, hi_body, 0)

  lane = lax.iota(jnp.int32, L)
  m127 = jnp.full((L,), 127, jnp.int32)

  for c in range(NCH):
    cps = [
        pltpu.async_copy(ue_r.at[idx_u.at[c]], ru, sem),
        pltpu.async_copy(ie_r.at[idx_i.at[c]], ri, sem),
        pltpu.async_copy(ub_r.at[idxb_u.at[c]], rbu, sem),
        pltpu.async_copy(ib_r.at[idxb_i.at[c]], rbi, sem),
    ]
    for cp in cps:
      cp.wait()

    def group_body(g, _, c=c):
      sl = pl.ds(g * L, L)
      acc = jnp.zeros((L,), jnp.float32)
      for u in range(L):
        r = g * L + u
        p = (ru[r, pl.ds(0, L)] * ri[r, pl.ds(0, L)]
             + ru[r, pl.ds(L, L)] * ri[r, pl.ds(L, L)])
        acc = jnp.where(lane == u, jnp.sum(p), acc)
      rows = g * L + lane
      bu_v = plsc.load_gather(rbu, [rows, idx_u[c, sl] & m127])
      bi_v = plsc.load_gather(rbi, [rows, idx_i[c, sl] & m127])
      x = acc + bu_v + bi_v
      preds[pl.ds(c * CHUNK + g * L, L)] = 1.0 / (1.0 + jnp.exp(-x))
      return 0
    lax.fori_loop(0, CHUNK // L, group_body, 0)

  pltpu.sync_copy(preds, out_r.at[pl.ds(base, BPW)])


@jax.jit
def kernel(user, item, user_e, item_e, user_b, item_b):
  user = user.astype(jnp.int32)
  item = item.astype(jnp.int32)

  padn = BIAS_R * 128 - NROWS
  ubp = jnp.pad(user_b, ((0, padn), (0, 0))).reshape(BIAS_R, 128)
  ibp = jnp.pad(item_b, ((0, padn), (0, 0))).reshape(BIAS_R, 128)

  mesh = plsc.VectorSubcoreMesh(core_axis_name="c", subcore_axis_name="s")
  run = pl.kernel(
      _mf_body,
      out_type=jax.ShapeDtypeStruct((B,), jnp.float32),
      mesh=mesh,
      compiler_params=pltpu.CompilerParams(
          needs_layout_passes=False, use_tc_tiling_on_sc=False),
      scratch_types=[
          pltpu.VMEM((NCH, CHUNK), jnp.int32),
          pltpu.VMEM((NCH, CHUNK), jnp.int32),
          pltpu.VMEM((NCH, CHUNK), jnp.int32),
          pltpu.VMEM((NCH, CHUNK), jnp.int32),
          pltpu.VMEM((CHUNK, D), jnp.float32),
          pltpu.VMEM((CHUNK, D), jnp.float32),
          pltpu.VMEM((CHUNK, 128), jnp.float32),
          pltpu.VMEM((CHUNK, 128), jnp.float32),
          pltpu.VMEM((BPW,), jnp.float32),
          pltpu.SemaphoreType.DMA,
      ],
  )
  return run(user, item, user_e, item_e, ubp, ibp)
